# Initial kernel scaffold; baseline (speedup 1.0000x reference)
import jax
import jax.numpy as jnp
from jax.experimental import pallas as pl


def kernel(dense_content_embedding, enc_W, enc_b, cb0, cb1, cb2, dec_W, dec_b):
    raise NotImplementedError("write your pallas kernel here")


# speedup vs baseline: 1.2564x; 1.2564x over previous
import functools

import jax
import jax.numpy as jnp
from jax import lax
from jax.experimental import pallas as pl
from jax.experimental.pallas import tpu as pltpu
from jax.experimental.pallas import tpu_sc as plsc

B = 4096
K = 8192
D = 256
IN_D = 768
BT = 256
NBT = B // BT
DEC_BT = 512


def _enc_dist_body(x_ref, w_ref, b_ref, cb_ref, idx_ref):
    r = lax.dot_general(x_ref[...], w_ref[...], (((1,), (0,)), ((), ())),
                        preferred_element_type=jnp.float32) + b_ref[...]
    r_sq = jnp.sum(r * r, axis=-1, keepdims=True)
    cross = lax.dot_general(r, cb_ref[...], (((1,), (1,)), ((), ())),
                            preferred_element_type=jnp.float32)
    c_sq = jnp.sum(cb_ref[...] * cb_ref[...], axis=-1)[None, :]
    dist = r_sq - 2.0 * cross + c_sq
    idx_ref[0, 0, :] = jnp.argmin(dist, axis=-1).astype(jnp.int32)


def _dist_body(q_ref, cb_ref, idx_ref):
    q = q_ref[...]
    r_sq = jnp.sum(q * q, axis=-1, keepdims=True)
    cross = lax.dot_general(q, cb_ref[...], (((1,), (1,)), ((), ())),
                            preferred_element_type=jnp.float32)
    c_sq = jnp.sum(cb_ref[...] * cb_ref[...], axis=-1)[None, :]
    dist = r_sq - 2.0 * cross + c_sq
    idx_ref[0, 0, :] = jnp.argmin(dist, axis=-1).astype(jnp.int32)


def _enc_argmin(x, enc_W, enc_b2, cb):
    return pl.pallas_call(
        _enc_dist_body,
        grid=(NBT,),
        in_specs=[
            pl.BlockSpec((BT, IN_D), lambda b: (b, 0)),
            pl.BlockSpec((IN_D, D), lambda b: (0, 0)),
            pl.BlockSpec((1, D), lambda b: (0, 0)),
            pl.BlockSpec((K, D), lambda b: (0, 0)),
        ],
        out_specs=pl.BlockSpec((1, 1, BT), lambda b: (b, 0, 0)),
        out_shape=jax.ShapeDtypeStruct((NBT, 1, BT), jnp.int32),
    )(x, enc_W, enc_b2, cb)


def _argmin(q, cb):
    return pl.pallas_call(
        _dist_body,
        grid=(NBT,),
        in_specs=[
            pl.BlockSpec((BT, D), lambda b: (b, 0)),
            pl.BlockSpec((K, D), lambda b: (0, 0)),
        ],
        out_specs=pl.BlockSpec((1, 1, BT), lambda b: (b, 0, 0)),
        out_shape=jax.ShapeDtypeStruct((NBT, 1, BT), jnp.int32),
    )(q, cb)


def _sc_gather(table, idx):
    info = plsc.get_sparse_core_info()
    nc, ns = info.num_cores, info.num_subcores
    nw = nc * ns
    b_per_w = B // nw
    mesh = plsc.VectorSubcoreMesh(core_axis_name="c", subcore_axis_name="s")

    @functools.partial(
        pl.kernel, mesh=mesh,
        out_type=jax.ShapeDtypeStruct((B, D), jnp.float32),
        scratch_types=[
            pltpu.VMEM((b_per_w,), jnp.int32),
            pltpu.VMEM((b_per_w, D), jnp.float32),
            pltpu.SemaphoreType.DMA,
        ],
    )
    def k(table_hbm, idx_hbm, out_hbm, idx_v, rows_v, sem):
        wid = lax.axis_index("s") * nc + lax.axis_index("c")
        base = wid * b_per_w
        pltpu.sync_copy(idx_hbm.at[pl.ds(base, b_per_w)], idx_v)
        pltpu.async_copy(table_hbm.at[idx_v], rows_v, sem).wait()
        pltpu.sync_copy(rows_v, out_hbm.at[pl.ds(base, b_per_w)])

    return k(table, idx)


def _decode_body(q0_ref, q1_ref, q2_ref, w_ref, b_ref, o_ref):
    s = (q0_ref[...] + q1_ref[...]) + q2_ref[...]
    o_ref[...] = lax.dot_general(s, w_ref[...], (((1,), (0,)), ((), ())),
                                 preferred_element_type=jnp.float32) + b_ref[...]


def _decode(q0, q1, q2, dec_W, dec_b2):
    return pl.pallas_call(
        _decode_body,
        grid=(B // DEC_BT,),
        in_specs=[
            pl.BlockSpec((DEC_BT, D), lambda b: (b, 0)),
            pl.BlockSpec((DEC_BT, D), lambda b: (b, 0)),
            pl.BlockSpec((DEC_BT, D), lambda b: (b, 0)),
            pl.BlockSpec((D, D), lambda b: (0, 0)),
            pl.BlockSpec((1, D), lambda b: (0, 0)),
        ],
        out_specs=pl.BlockSpec((DEC_BT, D), lambda b: (b, 0)),
        out_shape=jax.ShapeDtypeStruct((B, D), jnp.float32),
    )(q0, q1, q2, dec_W, dec_b2)


def kernel(dense_content_embedding, enc_W, enc_b, cb0, cb1, cb2, dec_W, dec_b):
    idx0 = _enc_argmin(dense_content_embedding, enc_W,
                       enc_b.reshape(1, D), cb0).reshape(B)
    q0 = _sc_gather(cb0, idx0)
    idx1 = _argmin(q0, cb1).reshape(B)
    q1 = _sc_gather(cb1, idx1)
    idx2 = _argmin(q1, cb2).reshape(B)
    q2 = _sc_gather(cb2, idx2)
    return _decode(q0, q1, q2, dec_W, dec_b.reshape(1, D))
